# baseline (speedup 1.0000x reference)
import functools

import jax
import jax.numpy as jnp
from jax import lax
from jax.experimental import pallas as pl
from jax.experimental.pallas import tpu as pltpu
from jax.experimental.pallas import tpu_sc as plsc

_VOCAB = 1000
_DIM = 128
_BATCH = 4096
_SEQ = 200
_NCLASS = 3
_CPAD = 8

_NC, _NS = 2, 16
_NW = _NC * _NS
_ROWS = _BATCH // _NW
_GROUPS = _ROWS // 16


def _table_body(emb_ref, w_ref, b_ref, out_ref):
    acc = jnp.dot(emb_ref[...], w_ref[...],
                  preferred_element_type=jnp.float32,
                  precision=lax.Precision.HIGHEST)
    out_ref[...] = (acc + b_ref[...]) * (1.0 / _SEQ)


def _make_table(emb, W, b):
    w_pad = jnp.zeros((_DIM, _CPAD), jnp.float32).at[:, :_NCLASS].set(W.T)
    b_pad = jnp.zeros((1, _CPAD), jnp.float32).at[0, :_NCLASS].set(b)
    return pl.pallas_call(
        _table_body,
        out_shape=jax.ShapeDtypeStruct((_VOCAB, _CPAD), jnp.float32),
    )(emb, w_pad, b_pad)


def _sc_body(x_hbm, t0_hbm, t1_hbm, t2_hbm,
             o0_hbm, o1_hbm, o2_hbm,
             xv, t0v, t1v, t2v, o0v, o1v, o2v):
    wid = lax.axis_index("s") * _NC + lax.axis_index("c")
    base = wid * _ROWS

    pltpu.sync_copy(x_hbm.at[pl.ds(base, _ROWS), :], xv)
    pltpu.sync_copy(t0_hbm, t0v)
    pltpu.sync_copy(t1_hbm, t1v)
    pltpu.sync_copy(t2_hbm, t2v)

    row_vecs = [lax.iota(jnp.int32, 16) + g * 16 for g in range(_GROUPS)]

    def lbody(l, accs):
        lvec = jnp.full((16,), 0, jnp.int32) + l
        out = []
        for g in range(_GROUPS):
            a0, a1, a2 = accs[g]
            idx = plsc.load_gather(xv, [row_vecs[g], lvec])
            a0 = a0 + plsc.load_gather(t0v, [idx])
            a1 = a1 + plsc.load_gather(t1v, [idx])
            a2 = a2 + plsc.load_gather(t2v, [idx])
            out.append((a0, a1, a2))
        return tuple(out)

    z = jnp.zeros((16,), jnp.float32)
    init = tuple((z, z, z) for _ in range(_GROUPS))
    accs = lax.fori_loop(0, _SEQ, lbody, init)

    for g in range(_GROUPS):
        a0, a1, a2 = accs[g]
        o0v[pl.ds(g * 16, 16)] = a0
        o1v[pl.ds(g * 16, 16)] = a1
        o2v[pl.ds(g * 16, 16)] = a2

    pltpu.sync_copy(o0v, o0_hbm.at[pl.ds(base, _ROWS)])
    pltpu.sync_copy(o1v, o1_hbm.at[pl.ds(base, _ROWS)])
    pltpu.sync_copy(o2v, o2_hbm.at[pl.ds(base, _ROWS)])


@functools.cache
def _sc_gather_reduce():
    col = jax.ShapeDtypeStruct((_BATCH,), jnp.float32)
    return pl.kernel(
        _sc_body,
        out_type=(col, col, col),
        mesh=plsc.VectorSubcoreMesh(core_axis_name="c", subcore_axis_name="s",
                                    num_cores=_NC, num_subcores=_NS),
        compiler_params=pltpu.CompilerParams(needs_layout_passes=False),
        scratch_types=(
            pltpu.VMEM((_ROWS, _SEQ), jnp.int32),
            pltpu.VMEM((_VOCAB,), jnp.float32),
            pltpu.VMEM((_VOCAB,), jnp.float32),
            pltpu.VMEM((_VOCAB,), jnp.float32),
            pltpu.VMEM((_ROWS,), jnp.float32),
            pltpu.VMEM((_ROWS,), jnp.float32),
            pltpu.VMEM((_ROWS,), jnp.float32),
        ),
    )


def kernel(x, emb, W, b):
    t = _make_table(emb, W, b)
    o0, o1, o2 = _sc_gather_reduce()(
        x.astype(jnp.int32), t[:, 0], t[:, 1], t[:, 2])
    return jnp.stack([o0, o1, o2], axis=1)


# speedup vs baseline: 1.5032x; 1.5032x over previous
import functools

import jax
import jax.numpy as jnp
from jax import lax
from jax.experimental import pallas as pl
from jax.experimental.pallas import tpu as pltpu
from jax.experimental.pallas import tpu_sc as plsc

_VOCAB = 1000
_DIM = 128
_BATCH = 4096
_SEQ = 200
_NCLASS = 3
_REP = 16
_TW = _NCLASS * _REP

_NC, _NS = 2, 16
_NW = _NC * _NS
_ROWS = _BATCH // _NW
_GROUPS = _ROWS // 16


def _table_body(emb_ref, wrep_ref, out_ref):
    out_ref[...] = jnp.dot(emb_ref[...], wrep_ref[...],
                           preferred_element_type=jnp.float32,
                           precision=lax.Precision.HIGHEST) * (1.0 / _SEQ)


def _make_table(emb, W):
    wrep = jnp.repeat(W, _REP, axis=0).T
    return pl.pallas_call(
        _table_body,
        out_shape=jax.ShapeDtypeStruct((_VOCAB, _TW), jnp.float32),
    )(emb, wrep)


def _sc_body(xt_hbm, tp_hbm, o0_hbm, o1_hbm, o2_hbm,
             xv, tpv, o0v, o1v, o2v, sem1, sem2):
    wid = lax.axis_index("s") * _NC + lax.axis_index("c")
    base = wid * _ROWS

    cp1 = pltpu.async_copy(xt_hbm.at[:, pl.ds(base, _ROWS)], xv, sem1)
    cp2 = pltpu.async_copy(tp_hbm, tpv, sem2)
    cp1.wait()
    cp2.wait()

    lane = lax.iota(jnp.int32, 16)
    t1 = tpv.at[pl.ds(_REP, _VOCAB * _TW - _REP)]
    t2 = tpv.at[pl.ds(2 * _REP, _VOCAB * _TW - 2 * _REP)]

    for g in range(_GROUPS):
        def lbody(l, accs):
            a0, a1, a2 = accs
            idx = xv[l, pl.ds(g * 16, 16)]
            pos = idx * _TW + lane
            a0 = a0 + plsc.load_gather(tpv, [pos])
            a1 = a1 + plsc.load_gather(t1, [pos])
            a2 = a2 + plsc.load_gather(t2, [pos])
            return (a0, a1, a2)

        z = jnp.zeros((16,), jnp.float32)
        a0, a1, a2 = lax.fori_loop(0, _SEQ, lbody, (z, z, z))
        o0v[pl.ds(g * 16, 16)] = a0
        o1v[pl.ds(g * 16, 16)] = a1
        o2v[pl.ds(g * 16, 16)] = a2

    pltpu.sync_copy(o0v, o0_hbm.at[pl.ds(base, _ROWS)])
    pltpu.sync_copy(o1v, o1_hbm.at[pl.ds(base, _ROWS)])
    pltpu.sync_copy(o2v, o2_hbm.at[pl.ds(base, _ROWS)])


@functools.cache
def _sc_gather_reduce():
    col = jax.ShapeDtypeStruct((_BATCH,), jnp.float32)
    return pl.kernel(
        _sc_body,
        out_type=(col, col, col),
        mesh=plsc.VectorSubcoreMesh(core_axis_name="c", subcore_axis_name="s",
                                    num_cores=_NC, num_subcores=_NS),
        compiler_params=pltpu.CompilerParams(needs_layout_passes=False),
        scratch_types=(
            pltpu.VMEM((_SEQ, _ROWS), jnp.int32),
            pltpu.VMEM((_VOCAB * _TW,), jnp.float32),
            pltpu.VMEM((_ROWS,), jnp.float32),
            pltpu.VMEM((_ROWS,), jnp.float32),
            pltpu.VMEM((_ROWS,), jnp.float32),
            pltpu.SemaphoreType.DMA,
            pltpu.SemaphoreType.DMA,
        ),
    )


def kernel(x, emb, W, b):
    tp = _make_table(emb, W).reshape(-1)
    xt = jnp.transpose(x.astype(jnp.int32))
    o0, o1, o2 = _sc_gather_reduce()(xt, tp)
    return jnp.stack([o0, o1, o2], axis=1) + b


# speedup vs baseline: 1.5650x; 1.0411x over previous
import functools

import jax
import jax.numpy as jnp
from jax import lax
from jax.experimental import pallas as pl
from jax.experimental.pallas import tpu as pltpu
from jax.experimental.pallas import tpu_sc as plsc

_VOCAB = 1000
_DIM = 128
_BATCH = 4096
_SEQ = 200
_NCLASS = 3
_REP = 16
_TW = _NCLASS * _REP

_NC, _NS = 2, 16
_NW = _NC * _NS
_ROWS = _BATCH // _NW
_GROUPS = _ROWS // 16


def _table_body(emb_ref, wrep_ref, out_ref):
    out_ref[...] = jnp.dot(emb_ref[...], wrep_ref[...],
                           preferred_element_type=jnp.float32,
                           precision=lax.Precision.HIGHEST) * (1.0 / _SEQ)


def _make_table(emb, W):
    wrep = jnp.repeat(W, _REP, axis=0).T
    return pl.pallas_call(
        _table_body,
        out_shape=jax.ShapeDtypeStruct((_VOCAB, _TW), jnp.float32),
    )(emb, wrep)


def _sc_body(xt_hbm, tp_hbm, o0_hbm, o1_hbm, o2_hbm,
             xv, tpv, o0v, o1v, o2v, sem1, sem2):
    wid = lax.axis_index("s") * _NC + lax.axis_index("c")
    base = wid * _ROWS

    cp1 = pltpu.async_copy(xt_hbm.at[:, pl.ds(base, _ROWS)], xv, sem1)
    cp2 = pltpu.async_copy(tp_hbm, tpv, sem2)
    cp1.wait()
    cp2.wait()

    lane = lax.iota(jnp.int32, 16)
    t1 = tpv.at[pl.ds(_REP, _VOCAB * _TW - _REP)]
    t2 = tpv.at[pl.ds(2 * _REP, _VOCAB * _TW - 2 * _REP)]

    for g in range(_GROUPS):
        def lbody(l, accs):
            a0, a1, a2 = accs
            idx = xv[l, pl.ds(g * 16, 16)]
            pos = idx * _TW + lane
            a0 = a0 + plsc.load_gather(tpv, [pos])
            a1 = a1 + plsc.load_gather(t1, [pos])
            a2 = a2 + plsc.load_gather(t2, [pos])
            return (a0, a1, a2)

        z = jnp.zeros((16,), jnp.float32)
        a0, a1, a2 = plsc.parallel_loop(
            0, _SEQ, 1, unroll=4, carry=(z, z, z))(lbody)
        o0v[pl.ds(g * 16, 16)] = a0
        o1v[pl.ds(g * 16, 16)] = a1
        o2v[pl.ds(g * 16, 16)] = a2

    pltpu.sync_copy(o0v, o0_hbm.at[pl.ds(base, _ROWS)])
    pltpu.sync_copy(o1v, o1_hbm.at[pl.ds(base, _ROWS)])
    pltpu.sync_copy(o2v, o2_hbm.at[pl.ds(base, _ROWS)])


@functools.cache
def _sc_gather_reduce():
    col = jax.ShapeDtypeStruct((_BATCH,), jnp.float32)
    return pl.kernel(
        _sc_body,
        out_type=(col, col, col),
        mesh=plsc.VectorSubcoreMesh(core_axis_name="c", subcore_axis_name="s",
                                    num_cores=_NC, num_subcores=_NS),
        compiler_params=pltpu.CompilerParams(needs_layout_passes=False),
        scratch_types=(
            pltpu.VMEM((_SEQ, _ROWS), jnp.int32),
            pltpu.VMEM((_VOCAB * _TW,), jnp.float32),
            pltpu.VMEM((_ROWS,), jnp.float32),
            pltpu.VMEM((_ROWS,), jnp.float32),
            pltpu.VMEM((_ROWS,), jnp.float32),
            pltpu.SemaphoreType.DMA,
            pltpu.SemaphoreType.DMA,
        ),
    )


def kernel(x, emb, W, b):
    tp = _make_table(emb, W).reshape(-1)
    xt = jnp.transpose(x.astype(jnp.int32))
    o0, o1, o2 = _sc_gather_reduce()(xt, tp)
    return jnp.stack([o0, o1, o2], axis=1) + b
